# baseline (speedup 1.0000x reference)
import functools

import jax
import jax.numpy as jnp
from jax import lax
from jax.experimental import pallas as pl
from jax.experimental.pallas import tpu as pltpu
from jax.experimental.pallas import tpu_sc as plsc

B = 128
N = 32768
BLOCK_ROWS = 64

_NUM_WORKERS = 32
_CHUNK_COLS = 4096
_COL_CHUNKS = N // _CHUNK_COLS
_N_CHUNKS = (B // 8) * _COL_CHUNKS
_CHUNKS_PER_WORKER = _N_CHUNKS // _NUM_WORKERS


def _softmax_block(x_ref, o_ref):
    x = x_ref[...]
    m = jnp.max(x, axis=1, keepdims=True)
    e = jnp.exp(x - m)
    s = jnp.sum(e, axis=1, keepdims=True)
    o_ref[...] = e * (1.0 / s)


def _tc_softmax(output_true):
    grid = (B // BLOCK_ROWS,)
    return pl.pallas_call(
        _softmax_block,
        grid=grid,
        in_specs=[pl.BlockSpec((BLOCK_ROWS, N), lambda i: (i, 0))],
        out_specs=pl.BlockSpec((BLOCK_ROWS, N), lambda i: (i, 0)),
        out_shape=jax.ShapeDtypeStruct((B, N), output_true.dtype),
    )(output_true)


@functools.partial(
    pl.kernel,
    mesh=plsc.VectorSubcoreMesh(core_axis_name="c", subcore_axis_name="s"),
    out_type=jax.ShapeDtypeStruct((B, N + 1), jnp.float32),
    scratch_types=[
        pltpu.VMEM((2, 8, _CHUNK_COLS), jnp.float32),
        pltpu.SemaphoreType.DMA((2,)),
    ],
)
def _sc_relayout(probs_hbm, out_hbm, buf, outsem):
    wid = lax.axis_index("s") * 2 + lax.axis_index("c")
    out_handles = {}
    for k in range(_CHUNKS_PER_WORKER):
        slot = k % 2
        if k >= 2:
            out_handles[k - 2].wait()
        chunk = wid * _CHUNKS_PER_WORKER + k
        a = chunk // _COL_CHUNKS
        c = chunk % _COL_CHUNKS
        rows = pl.ds(a * 8, 8)
        cols = pl.ds(c * _CHUNK_COLS, _CHUNK_COLS)
        pltpu.sync_copy(probs_hbm.at[rows, cols], buf.at[slot])
        out_handles[k] = pltpu.async_copy(
            buf.at[slot], out_hbm.at[rows, cols], outsem.at[slot]
        )
    out_handles[_CHUNKS_PER_WORKER - 2].wait()
    out_handles[_CHUNKS_PER_WORKER - 1].wait()


def _zero_col_body(o_in_ref, o_ref, zcol, sem):
    del o_in_ref
    zcol[...] = jnp.zeros_like(zcol)
    pltpu.make_async_copy(zcol, o_ref.at[:, pl.ds(N, 1)], sem).start()
    pltpu.make_async_copy(zcol, o_ref.at[:, pl.ds(N, 1)], sem).wait()


def _tc_zero_col(out):
    return pl.pallas_call(
        _zero_col_body,
        in_specs=[pl.BlockSpec(memory_space=pl.ANY)],
        out_specs=pl.BlockSpec(memory_space=pl.ANY),
        out_shape=jax.ShapeDtypeStruct((B, N + 1), jnp.float32),
        input_output_aliases={0: 0},
        scratch_shapes=[
            pltpu.VMEM((B, 1), jnp.float32),
            pltpu.SemaphoreType.DMA,
        ],
    )(out)


def kernel(output_true):
    probs = _tc_softmax(output_true)
    out = _sc_relayout(probs)
    return _tc_zero_col(out)


# speedup vs baseline: 1.9621x; 1.9621x over previous
import jax
import jax.numpy as jnp
from jax.experimental import pallas as pl
from jax.experimental.pallas import tpu as pltpu

B = 128
N = 32768
BLOCK_ROWS = 32
GRID = B // BLOCK_ROWS


def _softmax_block(x_ref, o_hbm, scratch, zcol, sems, zsem):
    i = pl.program_id(0)

    @pl.when(i == 0)
    def _zero_col():
        zcol[...] = jnp.zeros_like(zcol)
        pltpu.make_async_copy(zcol, o_hbm.at[:, pl.ds(N, 1)], zsem).start()

    x = x_ref[...]
    m = jnp.max(x, axis=1, keepdims=True)
    e = jnp.exp(x - m)
    s = jnp.sum(e, axis=1, keepdims=True)
    scratch[i] = e * (1.0 / s)

    pltpu.make_async_copy(
        scratch.at[i],
        o_hbm.at[pl.ds(i * BLOCK_ROWS, BLOCK_ROWS), pl.ds(0, N)],
        sems.at[i],
    ).start()

    @pl.when(i == GRID - 1)
    def _drain():
        for k in range(GRID):
            pltpu.make_async_copy(
                scratch.at[k],
                o_hbm.at[pl.ds(k * BLOCK_ROWS, BLOCK_ROWS), pl.ds(0, N)],
                sems.at[k],
            ).wait()
        pltpu.make_async_copy(zcol, o_hbm.at[:, pl.ds(N, 1)], zsem).wait()


def kernel(output_true):
    return pl.pallas_call(
        _softmax_block,
        grid=(GRID,),
        in_specs=[pl.BlockSpec((BLOCK_ROWS, N), lambda i: (i, 0))],
        out_specs=pl.BlockSpec(memory_space=pl.ANY),
        out_shape=jax.ShapeDtypeStruct((B, N + 1), output_true.dtype),
        scratch_shapes=[
            pltpu.VMEM((GRID, BLOCK_ROWS, N), jnp.float32),
            pltpu.VMEM((B, 1), jnp.float32),
            pltpu.SemaphoreType.DMA((GRID,)),
            pltpu.SemaphoreType.DMA,
        ],
    )(output_true)


# speedup vs baseline: 2.0811x; 1.0607x over previous
import jax
import jax.numpy as jnp
from jax.experimental import pallas as pl

B = 128
N = 32768
BLOCK_ROWS = 64


def _softmax_block(x_ref, o_ref):
    x = x_ref[...]
    m = jnp.max(x, axis=1, keepdims=True)
    e = jnp.exp(x - m)
    s = jnp.sum(e, axis=1, keepdims=True)
    o_ref[:, :N] = e * (1.0 / s)
    o_ref[:, N:] = jnp.zeros((x.shape[0], 1), x.dtype)


def kernel(output_true):
    grid = (B // BLOCK_ROWS,)
    return pl.pallas_call(
        _softmax_block,
        grid=grid,
        in_specs=[pl.BlockSpec((BLOCK_ROWS, N), lambda i: (i, 0))],
        out_specs=pl.BlockSpec((BLOCK_ROWS, N + 1), lambda i: (i, 0)),
        out_shape=jax.ShapeDtypeStruct((B, N + 1), output_true.dtype),
    )(output_true)
